# baseline (speedup 1.0000x reference)
import functools

import jax
import jax.numpy as jnp
from jax import lax
from jax.experimental import pallas as pl
from jax.experimental.pallas import tpu as pltpu
from jax.experimental.pallas import tpu_sc as plsc

N0, N1 = 64, 64
D0, D1 = 1024, 1024
NC, NS = 2, 16
NW = NC * NS
B = 256
BPW = B // NW
REP = 32
NBUF = 3
WPB = N0 // REP
GT = 16

_mesh = plsc.VectorSubcoreMesh(core_axis_name="c", subcore_axis_name="s")


@functools.partial(
    pl.kernel,
    mesh=_mesh,
    out_type=jax.ShapeDtypeStruct((B * N0, D0 + D1), jnp.float32),
    scratch_types=[
        pltpu.VMEM((BPW, REP), jnp.int32),
        *[pltpu.VMEM((REP, D1), jnp.float32) for _ in range(NBUF)],
        *[pltpu.SemaphoreType.DMA for _ in range(NBUF)],
        *[pltpu.SemaphoreType.DMA for _ in range(NBUF)],
    ],
)
def _sc_gather(idx_hbm, w1_hbm, out_hbm, idx_v, *scr):
    bufs = scr[:NBUF]
    gsem = scr[NBUF:2 * NBUF]
    osem = scr[2 * NBUF:3 * NBUF]
    wid = lax.axis_index("s") * NC + lax.axis_index("c")
    base = wid * BPW
    pltpu.sync_copy(idx_hbm.at[pl.ds(base, BPW)], idx_v)

    gc = [None] * BPW
    wc = [None] * BPW
    for b in range(NBUF):
        gc[b] = pltpu.async_copy(w1_hbm.at[idx_v.at[b]], bufs[b], gsem[b])
    for b in range(BPW):
        x = b % NBUF
        row0 = (base + b) * N0
        gc[b].wait()
        wc[b] = tuple(
            pltpu.async_copy(
                bufs[x],
                out_hbm.at[pl.ds(row0 + r * REP, REP), pl.ds(D0, D1)],
                osem[x],
            )
            for r in range(WPB)
        )
        if b + NBUF < BPW:
            for c in wc[b]:
                c.wait()
            gc[b + NBUF] = pltpu.async_copy(
                w1_hbm.at[idx_v.at[b + NBUF]], bufs[x], gsem[x]
            )
    for b in range(BPW - NBUF, BPW):
        for c in wc[b]:
            c.wait()


def _tc_dense_body(w0_ref, acc_ref, out_ref):
    del acc_ref
    out_ref[...] = jnp.broadcast_to(w0_ref[...][None], (GT, N0, D0))


def kernel(position_ids, w0, w1):
    pid = position_ids.reshape(-1).astype(jnp.int32)
    idx_rep = jnp.broadcast_to(pid[:, None], (B, REP))
    half = _sc_gather(idx_rep, w1.reshape(N1, D1))
    half = half.reshape(B, N0, D0 + D1)
    out = pl.pallas_call(
        _tc_dense_body,
        grid=(B // GT,),
        in_specs=[
            pl.BlockSpec((N0, D0), lambda i: (0, 0)),
            pl.BlockSpec(memory_space=pl.ANY),
        ],
        out_specs=pl.BlockSpec((GT, N0, D0), lambda i: (i, 0, 0)),
        out_shape=jax.ShapeDtypeStruct((B, N0, D0 + D1), jnp.float32),
        input_output_aliases={1: 0},
    )(w0.reshape(N0, D0), half)
    return out.reshape(*position_ids.shape, N0, D0 + D1)


# speedup vs baseline: 1.4581x; 1.4581x over previous
import functools

import jax
import jax.numpy as jnp
from jax import lax
from jax.experimental import pallas as pl
from jax.experimental.pallas import tpu as pltpu
from jax.experimental.pallas import tpu_sc as plsc

N0, N1 = 64, 64
D0, D1 = 1024, 1024
NC, NS = 2, 16
NW = NC * NS
B = 256
BPW = B // NW
REP = 8
WPB = N0 // REP
GT = 16

_mesh = plsc.VectorSubcoreMesh(core_axis_name="c", subcore_axis_name="s")


@functools.partial(
    pl.kernel,
    mesh=_mesh,
    out_type=jax.ShapeDtypeStruct((B * N0, D0 + D1), jnp.float32),
    scratch_types=[
        pltpu.VMEM((BPW, REP), jnp.int32),
        *[pltpu.VMEM((REP, D1), jnp.float32) for _ in range(BPW)],
        *[pltpu.SemaphoreType.DMA for _ in range(BPW)],
        pltpu.SemaphoreType.DMA,
    ],
)
def _sc_gather(idx_hbm, w1_hbm, out_hbm, idx_v, *scr):
    bufs = scr[:BPW]
    gsem = scr[BPW:2 * BPW]
    osem = scr[2 * BPW]
    wid = lax.axis_index("s") * NC + lax.axis_index("c")
    base = wid * BPW
    pltpu.sync_copy(idx_hbm.at[pl.ds(base, BPW)], idx_v)

    gc = [
        pltpu.async_copy(w1_hbm.at[idx_v.at[b]], bufs[b], gsem[b])
        for b in range(BPW)
    ]
    wc = []
    for b in range(BPW):
        row0 = (base + b) * N0
        gc[b].wait()
        wc.extend(
            pltpu.async_copy(
                bufs[b],
                out_hbm.at[pl.ds(row0 + r * REP, REP), pl.ds(D0, D1)],
                osem,
            )
            for r in range(WPB)
        )
    for c in wc:
        c.wait()


def _tc_dense_body(w0_ref, acc_ref, out_ref):
    del acc_ref
    out_ref[...] = jnp.broadcast_to(w0_ref[...][None], (GT, N0, D0))


def kernel(position_ids, w0, w1):
    pid = position_ids.reshape(-1).astype(jnp.int32)
    idx_rep = jnp.broadcast_to(pid[:, None], (B, REP))
    half = _sc_gather(idx_rep, w1.reshape(N1, D1))
    half = half.reshape(B, N0, D0 + D1)
    out = pl.pallas_call(
        _tc_dense_body,
        grid=(B // GT,),
        in_specs=[
            pl.BlockSpec((N0, D0), lambda i: (0, 0)),
            pl.BlockSpec(memory_space=pl.ANY),
        ],
        out_specs=pl.BlockSpec((GT, N0, D0), lambda i: (i, 0, 0)),
        out_shape=jax.ShapeDtypeStruct((B, N0, D0 + D1), jnp.float32),
        input_output_aliases={1: 0},
    )(w0.reshape(N0, D0), half)
    return out.reshape(*position_ids.shape, N0, D0 + D1)


# speedup vs baseline: 1.8729x; 1.2844x over previous
import functools

import jax
import jax.numpy as jnp
from jax import lax
from jax.experimental import pallas as pl
from jax.experimental.pallas import tpu as pltpu
from jax.experimental.pallas import tpu_sc as plsc

N0, N1 = 64, 64
D0, D1 = 1024, 1024
NC, NS = 2, 16
NW = NC * NS
B = 256
BPW = B // NW
GT = 16

_mesh = plsc.VectorSubcoreMesh(core_axis_name="c", subcore_axis_name="s")


@functools.partial(
    pl.kernel,
    mesh=_mesh,
    out_type=jax.ShapeDtypeStruct((B, D1), jnp.float32),
    scratch_types=[
        pltpu.VMEM((BPW,), jnp.int32),
        pltpu.VMEM((BPW, D1), jnp.float32),
        pltpu.SemaphoreType.DMA,
    ],
)
def _sc_gather(idx_hbm, w1_hbm, out_hbm, idx_v, rows_v, sem):
    wid = lax.axis_index("s") * NC + lax.axis_index("c")
    base = wid * BPW
    pltpu.sync_copy(idx_hbm.at[pl.ds(base, BPW)], idx_v)
    pltpu.async_copy(w1_hbm.at[idx_v], rows_v, sem).wait()
    pltpu.sync_copy(rows_v, out_hbm.at[pl.ds(base, BPW)])


def _tc_body(w0_ref, g_ref, out_ref):
    for g in range(GT):
        out_ref[g, :, :D0] = w0_ref[...]
        out_ref[g, :, D0:] = jnp.broadcast_to(g_ref[g][None], (N0, D1))


def kernel(position_ids, w0, w1):
    pid = position_ids.reshape(-1).astype(jnp.int32)
    rows = _sc_gather(pid, w1.reshape(N1, D1))
    out = pl.pallas_call(
        _tc_body,
        grid=(B // GT,),
        in_specs=[
            pl.BlockSpec((N0, D0), lambda i: (0, 0)),
            pl.BlockSpec((GT, D1), lambda i: (i, 0)),
        ],
        out_specs=pl.BlockSpec((GT, N0, D0 + D1), lambda i: (i, 0, 0)),
        out_shape=jax.ShapeDtypeStruct((B, N0, D0 + D1), jnp.float32),
    )(w0.reshape(N0, D0), rows)
    return out.reshape(*position_ids.shape, N0, D0 + D1)
